# Initial kernel scaffold; baseline (speedup 1.0000x reference)
import jax
import jax.numpy as jnp
from jax.experimental import pallas as pl


def kernel(input_ids, state_embeddings, superposition_matrix):
    raise NotImplementedError("write your pallas kernel here")


# speedup vs baseline: 46.5843x; 46.5843x over previous
import functools

import jax
import jax.numpy as jnp
from jax import lax
from jax.experimental import pallas as pl
from jax.experimental.pallas import tpu as pltpu
from jax.experimental.pallas import tpu_sc as plsc

_VTILE = 1000


def _mix_body(t_ref, w_ref, o_ref):
    o_ref[...] = jnp.dot(t_ref[...], w_ref[...],
                         preferred_element_type=jnp.float32)


def _precompute_table(table_flat, w):
    V, NE = table_flat.shape
    E = w.shape[1]
    return pl.pallas_call(
        _mix_body,
        grid=(V // _VTILE,),
        in_specs=[pl.BlockSpec((_VTILE, NE), lambda i: (i, 0)),
                  pl.BlockSpec((NE, E), lambda i: (0, 0))],
        out_specs=pl.BlockSpec((_VTILE, E), lambda i: (i, 0)),
        out_shape=jax.ShapeDtypeStruct((V, E), jnp.float32),
    )(table_flat, w)


@functools.cache
def _make_sc_gather(V, N, D):
    info = plsc.get_sparse_core_info()
    NC, NS = info.num_cores, info.num_subcores
    NW = NC * NS
    n_per_w = N // NW
    mesh = plsc.VectorSubcoreMesh(core_axis_name="c", subcore_axis_name="s")

    @functools.partial(
        pl.kernel, mesh=mesh,
        out_type=jax.ShapeDtypeStruct((N, D), jnp.float32),
        scratch_types=[
            pltpu.VMEM((n_per_w,), jnp.int32),
            pltpu.VMEM((n_per_w, D), jnp.float32),
            pltpu.SemaphoreType.DMA,
        ],
        compiler_params=pltpu.CompilerParams(use_tc_tiling_on_sc=False),
    )
    def k(table_hbm, idx_hbm, out_hbm, idx_v, rows_v, sem):
        wid = lax.axis_index("s") * NC + lax.axis_index("c")
        base = wid * n_per_w
        pltpu.sync_copy(idx_hbm.at[pl.ds(base, n_per_w)], idx_v)
        pltpu.async_copy(table_hbm.at[idx_v], rows_v, sem).wait()
        pltpu.sync_copy(rows_v, out_hbm.at[pl.ds(base, n_per_w)])

    return k


def kernel(input_ids, state_embeddings, superposition_matrix):
    V, NSt, E = state_embeddings.shape
    Bb, Ss = input_ids.shape
    w = jnp.tile(superposition_matrix * (1.0 / NSt), (NSt, 1))
    table2 = _precompute_table(state_embeddings.reshape(V, NSt * E), w)
    flat_ids = input_ids.reshape(-1).astype(jnp.int32)
    out = _make_sc_gather(V, flat_ids.shape[0], E)(table2, flat_ids)
    return out.reshape(Bb, Ss, E)
